# baseline (speedup 1.0000x reference)
import jax
import jax.numpy as jnp
from jax import lax
from jax.experimental import pallas as pl
from jax.experimental.pallas import tpu as pltpu

_E = 16
_ROWS = 8192
_COLS = 2048
_BLK = 512
_GRID = _ROWS // (2 * _BLK)
_INV_N = 1.0 / float(_ROWS * _COLS)
_ALPHA = 1.0 / 1000.0
_Z = 0.001


def _router_kernel(a_ref, b_ref, proto_ref, ema_ref,
                   mask_ref, probs_ref, loss_ref, idx_ref, usage_ref,
                   acc_ref):
    i = pl.program_id(0)

    @pl.when(i == 0)
    def _init():
        acc_ref[0] = 0.0

    acc_ref[0] += jnp.sum(a_ref[...]) + jnp.sum(b_ref[...])

    @pl.when(i == _GRID - 1)
    def _epilogue():
        x = acc_ref[0] * _INV_N
        ids = lax.broadcasted_iota(jnp.int32, (1, _E), 1)
        p = proto_ref[...]
        sim = -((p - x) ** 2)
        m = jnp.max(sim)
        e = jnp.exp(sim - m)
        probs = e / jnp.sum(e)
        m1 = jnp.max(probs)
        i1 = jnp.min(jnp.where(probs == m1, ids, _E))
        hit1 = ids == i1
        probs2 = jnp.where(hit1, -jnp.inf, probs)
        m2 = jnp.max(probs2)
        i2 = jnp.min(jnp.where(probs2 == m2, ids, _E))
        mask = (hit1 | (ids == i2)).astype(jnp.float32)
        usage = (1.0 - _ALPHA) * ema_ref[...] + _ALPHA * mask
        d = usage - (1.0 / _E)
        loss = jnp.sum(d * d) * (1.0 / _E) * _Z
        mask_ref[...] = mask
        probs_ref[...] = probs
        loss_ref[...] = jnp.full((1, _E), loss, jnp.float32)
        idx_ref[...] = jnp.where(ids == 0, i1, jnp.where(ids == 1, i2, 0))
        usage_ref[...] = usage


@jax.jit
def kernel(wm_state, prototypes, usage_ema):
    wm3d = wm_state.reshape(_ROWS // _BLK, _BLK, _COLS)
    full = pl.BlockSpec((1, _E), lambda i: (0, 0))
    outs = pl.pallas_call(
        _router_kernel,
        grid=(_GRID,),
        in_specs=[
            pl.BlockSpec((_BLK, _COLS), lambda i: (2 * i, 0)),
            pl.BlockSpec((1, _BLK, _COLS), lambda i: (2 * i + 1, 0, 0)),
            full,
            full,
        ],
        out_specs=[full, full, full, full, full],
        out_shape=[
            jax.ShapeDtypeStruct((1, _E), jnp.float32),
            jax.ShapeDtypeStruct((1, _E), jnp.float32),
            jax.ShapeDtypeStruct((1, _E), jnp.float32),
            jax.ShapeDtypeStruct((1, _E), jnp.int32),
            jax.ShapeDtypeStruct((1, _E), jnp.float32),
        ],
        scratch_shapes=[pltpu.SMEM((1,), jnp.float32)],
    )(wm_state, wm3d, prototypes.reshape(1, _E), usage_ema.reshape(1, _E))
    mask2d, probs2d, loss2d, idx2d, usage2d = outs
    return (mask2d[0], probs2d[0], loss2d[0, 0], idx2d[0, :2], usage2d[0])
